# baseline (speedup 1.0000x reference)
import functools
import math

import jax
import jax.numpy as jnp
from jax import lax
from jax.experimental import pallas as pl
from jax.experimental.pallas import tpu as pltpu
from jax.experimental.pallas import tpu_sc as plsc

VOCAB_N = 1000000
D = 64
SCALE = math.sqrt(D)

NUM_CORES = 2
NUM_SUBCORES = 16
NW = NUM_CORES * NUM_SUBCORES

TOTAL = 4096 * 200
PER_W = TOTAL // NW
CHUNK = 512
K = CHUNK // 128
N_CHUNKS = PER_W // CHUNK


def _body(x_hbm, table_hbm, out_hbm, idx_v, rows_v, gsem):
    wid = lax.axis_index("s") * NUM_CORES + lax.axis_index("c")
    chunk0 = wid * N_CHUNKS
    out0 = wid * PER_W

    def chunk_body(g, carry):
        pltpu.sync_copy(x_hbm.at[chunk0 + g], idx_v)
        copies = [
            pltpu.async_copy(
                table_hbm.at[idx_v.at[j]],
                rows_v.at[pl.ds(j * 128, 128)],
                gsem,
            )
            for j in range(K)
        ]
        for c in copies:
            c.wait()

        def scale_row(i, c2):
            for j in range(D // 16):
                sl = pl.ds(j * 16, 16)
                rows_v[i, sl] = rows_v[i, sl] * SCALE
            return c2

        lax.fori_loop(0, CHUNK, scale_row, 0, unroll=4)

        pltpu.sync_copy(rows_v, out_hbm.at[pl.ds(out0 + g * CHUNK, CHUNK)])
        return carry

    lax.fori_loop(0, N_CHUNKS, chunk_body, 0)


@functools.partial(jax.jit, donate_argnums=())
def kernel(x, table):
    x2 = x.reshape(NW * N_CHUNKS, K, 128).astype(jnp.int32)
    mesh = plsc.VectorSubcoreMesh(
        core_axis_name="c", subcore_axis_name="s",
        num_cores=NUM_CORES, num_subcores=NUM_SUBCORES,
    )
    run = pl.kernel(
        _body,
        out_type=jax.ShapeDtypeStruct((TOTAL, D), jnp.float32),
        mesh=mesh,
        scratch_types=[
            pltpu.VMEM((K, 128), jnp.int32),
            pltpu.VMEM((CHUNK, D), jnp.float32),
            pltpu.SemaphoreType.DMA,
        ],
        compiler_params=pltpu.CompilerParams(use_tc_tiling_on_sc=False),
    )
    out = run(x2, table)
    return out.reshape(x.shape[0], x.shape[1], D)


# speedup vs baseline: 1.0748x; 1.0748x over previous
import functools
import math

import jax
import jax.numpy as jnp
from jax import lax
from jax.experimental import pallas as pl
from jax.experimental.pallas import tpu as pltpu
from jax.experimental.pallas import tpu_sc as plsc

VOCAB_N = 1000000
D = 64
SCALE = math.sqrt(D)

NUM_CORES = 2
NUM_SUBCORES = 16
NW = NUM_CORES * NUM_SUBCORES

TOTAL = 4096 * 200
PER_W = TOTAL // NW
CHUNK = 512
K = CHUNK // 128
N_CHUNKS = PER_W // CHUNK


def _body(x_hbm, table_hbm, out_hbm,
          idx0, idx1, rows0, rows1, gsem0, gsem1, ssem0, ssem1):
    wid = lax.axis_index("s") * NUM_CORES + lax.axis_index("c")
    chunk0 = wid * N_CHUNKS
    out0 = wid * PER_W

    idx = (idx0, idx1)
    rows = (rows0, rows1)
    gsem = (gsem0, gsem1)
    ssem = (ssem0, ssem1)

    def fire_gather(g, b):
        pltpu.sync_copy(x_hbm.at[chunk0 + g], idx[b])
        for j in range(K):
            pltpu.async_copy(
                table_hbm.at[idx[b].at[j]],
                rows[b].at[pl.ds(j * 128, 128)],
                gsem[b],
            )

    def wait_gather(b):
        for j in range(K):
            pltpu.make_async_copy(
                table_hbm.at[idx[b].at[j]],
                rows[b].at[pl.ds(j * 128, 128)],
                gsem[b],
            ).wait()

    def out_slice(g):
        return out_hbm.at[pl.ds(out0 + g * CHUNK, CHUNK)]

    def scale_buf(b):
        def scale_row(i, c):
            for j in range(D // 16):
                sl = pl.ds(j * 16, 16)
                rows[b][i, sl] = rows[b][i, sl] * SCALE
            return c
        lax.fori_loop(0, CHUNK, scale_row, 0, unroll=8)

    def step(g, b):
        @pl.when(g > 0)
        def _():
            pltpu.make_async_copy(rows[1 - b], out_slice(g - 1), ssem[1 - b]).wait()

        @pl.when(g + 1 < N_CHUNKS)
        def _():
            fire_gather(g + 1, 1 - b)

        wait_gather(b)
        scale_buf(b)
        pltpu.async_copy(rows[b], out_slice(g), ssem[b])

    fire_gather(0, 0)

    def loop_body(i, c):
        step(2 * i, 0)
        step(2 * i + 1, 1)
        return c

    lax.fori_loop(0, N_CHUNKS // 2, loop_body, 0)

    pltpu.make_async_copy(rows[1], out_slice(N_CHUNKS - 1), ssem[1]).wait()


@functools.partial(jax.jit, donate_argnums=())
def kernel(x, table):
    x2 = x.reshape(NW * N_CHUNKS, K, 128).astype(jnp.int32)
    mesh = plsc.VectorSubcoreMesh(
        core_axis_name="c", subcore_axis_name="s",
        num_cores=NUM_CORES, num_subcores=NUM_SUBCORES,
    )
    run = pl.kernel(
        _body,
        out_type=jax.ShapeDtypeStruct((TOTAL, D), jnp.float32),
        mesh=mesh,
        scratch_types=[
            pltpu.VMEM((K, 128), jnp.int32),
            pltpu.VMEM((K, 128), jnp.int32),
            pltpu.VMEM((CHUNK, D), jnp.float32),
            pltpu.VMEM((CHUNK, D), jnp.float32),
            pltpu.SemaphoreType.DMA,
            pltpu.SemaphoreType.DMA,
            pltpu.SemaphoreType.DMA,
            pltpu.SemaphoreType.DMA,
        ],
        compiler_params=pltpu.CompilerParams(use_tc_tiling_on_sc=False),
    )
    out = run(x2, table)
    return out.reshape(x.shape[0], x.shape[1], D)
